# baseline (speedup 1.0000x reference)
import jax
import jax.numpy as jnp
from jax.experimental import pallas as pl
from jax.experimental.pallas import tpu as pltpu
from jax.experimental.pallas import tpu_sc as plsc

_CH = 512
_K = 8


def _sc_bias(hourly_table, daily_table):
    W = hourly_table.shape[1]
    n_active = W // 16
    mesh = plsc.VectorSubcoreMesh(core_axis_name="c", subcore_axis_name="s")

    def body(h_hbm, d_hbm, o_hbm, h_v, d_v):
        wid = jax.lax.axis_index("s") * 2 + jax.lax.axis_index("c")

        @pl.when(wid < n_active)
        def _():
            sl = pl.ds(wid * 16, 16)
            pltpu.sync_copy(h_hbm.at[0, sl], h_v)
            pltpu.sync_copy(d_hbm.at[0, sl], d_v)
            h_v[...] = h_v[...] + d_v[...]
            pltpu.sync_copy(h_v, o_hbm.at[0, sl])

    return pl.kernel(
        body,
        mesh=mesh,
        out_type=jax.ShapeDtypeStruct((1, W), jnp.float32),
        scratch_types=[
            pltpu.VMEM((16,), jnp.float32),
            pltpu.VMEM((16,), jnp.float32),
        ],
    )(hourly_table, daily_table)


def _make_body(B, S, D, CH, K):
    NC = S // CH
    T = B * NC
    W = D // 4

    def body(x_hbm, pe_hbm, bias_hbm, o_hbm,
             x_buf, o_buf, pe_buf, bias_buf,
             x_sem, o_sem, pe_sem, b_sem):
        def x_copy(t, slot):
            i, b = t // B, t % B
            return pltpu.make_async_copy(
                x_hbm.at[b, pl.ds(i * CH, CH), :], x_buf.at[slot], x_sem.at[slot])

        def o_copy(t, slot):
            i, b = t // B, t % B
            return pltpu.make_async_copy(
                o_buf.at[slot], o_hbm.at[b, pl.ds(i * CH, CH), :], o_sem.at[slot])

        def pe_copy(j):
            return pltpu.make_async_copy(
                pe_hbm.at[0, pl.ds(j * CH, CH), :],
                pe_buf.at[pl.ds(j * CH, CH), :], pe_sem.at[j])

        b_cp = pltpu.make_async_copy(bias_hbm, bias_buf, b_sem)
        b_cp.start()
        for j in range(NC):
            pe_copy(j).start()
        for t in range(min(K, T)):
            x_copy(t, t % K).start()
        b_cp.wait()
        bias = bias_buf[0, :]

        for t in range(T):
            i, b = t // B, t % B
            slot = t % K
            x_copy(t, slot).wait()
            if t >= K:
                o_copy(t - K, slot).wait()
            if b == 0:
                pe_copy(i).wait()
            for k in range(4):
                sl = slice(k * W, (k + 1) * W)
                o_buf[slot, :, sl] = x_buf[slot, :, sl] + (
                    pe_buf[pl.ds(i * CH, CH), sl] + bias[None, :])
            o_copy(t, slot).start()
            if t + K < T:
                x_copy(t + K, slot).start()
        for t in range(max(T - K, 0), T):
            o_copy(t, t % K).wait()

    return body


def kernel(x, timestamps, pe, hourly_table, daily_table):
    B, S, D = x.shape
    bias_row = _sc_bias(hourly_table, daily_table)
    ch = _CH if S % _CH == 0 else S
    body = _make_body(B, S, D, ch, _K)
    hbm = pl.BlockSpec(memory_space=pltpu.MemorySpace.HBM)
    return pl.pallas_call(
        body,
        in_specs=[hbm, hbm, hbm],
        out_specs=hbm,
        out_shape=jax.ShapeDtypeStruct((B, S, D), x.dtype),
        scratch_shapes=[
            pltpu.VMEM((_K, ch, D), x.dtype),
            pltpu.VMEM((_K, ch, D), x.dtype),
            pltpu.VMEM((S, D), pe.dtype),
            pltpu.VMEM((1, D // 4), jnp.float32),
            pltpu.SemaphoreType.DMA((_K,)),
            pltpu.SemaphoreType.DMA((_K,)),
            pltpu.SemaphoreType.DMA((S // ch,)),
            pltpu.SemaphoreType.DMA,
        ],
    )(x, pe, bias_row)


# speedup vs baseline: 1.8659x; 1.8659x over previous
import jax
import jax.numpy as jnp
from jax.experimental import pallas as pl
from jax.experimental.pallas import tpu as pltpu

_CH = 512
_K = 10


def _make_body(B, S, D, CH, K):
    NC = S // CH
    T = B * NC
    W = D // 4

    def body(x_hbm, pe_hbm, h_hbm, d_hbm, o_hbm,
             x_buf, o_buf, pe_buf, h_buf, d_buf,
             x_sem, o_sem, pe_sem, tbl_sem):
        def x_copy(t, slot):
            i, b = t // B, t % B
            return pltpu.make_async_copy(
                x_hbm.at[b, pl.ds(i * CH, CH), :], x_buf.at[slot], x_sem.at[slot])

        def o_copy(t, slot):
            i, b = t // B, t % B
            return pltpu.make_async_copy(
                o_buf.at[slot], o_hbm.at[b, pl.ds(i * CH, CH), :], o_sem.at[slot])

        def pe_copy(j):
            return pltpu.make_async_copy(
                pe_hbm.at[0, pl.ds(j * CH, CH), :],
                pe_buf.at[pl.ds(j * CH, CH), :], pe_sem.at[j])

        h_cp = pltpu.make_async_copy(h_hbm, h_buf, tbl_sem.at[0])
        d_cp = pltpu.make_async_copy(d_hbm, d_buf, tbl_sem.at[1])
        h_cp.start()
        d_cp.start()
        for j in range(NC):
            pe_copy(j).start()
        for t in range(min(K, T)):
            x_copy(t, t % K).start()
        h_cp.wait()
        d_cp.wait()
        bias = h_buf[0, :] + d_buf[0, :]

        for t in range(T):
            i, b = t // B, t % B
            slot = t % K
            x_copy(t, slot).wait()
            if t >= K:
                o_copy(t - K, slot).wait()
            if b == 0:
                pe_copy(i).wait()
            for k in range(4):
                sl = slice(k * W, (k + 1) * W)
                o_buf[slot, :, sl] = x_buf[slot, :, sl] + (
                    pe_buf[pl.ds(i * CH, CH), sl] + bias[None, :])
            o_copy(t, slot).start()
            if t + K < T:
                x_copy(t + K, slot).start()
        for t in range(max(T - K, 0), T):
            o_copy(t, t % K).wait()

    return body


def kernel(x, timestamps, pe, hourly_table, daily_table):
    B, S, D = x.shape
    ch = _CH if S % _CH == 0 else S
    body = _make_body(B, S, D, ch, _K)
    hbm = pl.BlockSpec(memory_space=pltpu.MemorySpace.HBM)
    return pl.pallas_call(
        body,
        in_specs=[hbm, hbm, hbm, hbm],
        out_specs=hbm,
        out_shape=jax.ShapeDtypeStruct((B, S, D), x.dtype),
        scratch_shapes=[
            pltpu.VMEM((_K, ch, D), x.dtype),
            pltpu.VMEM((_K, ch, D), x.dtype),
            pltpu.VMEM((S, D), pe.dtype),
            pltpu.VMEM(hourly_table.shape, hourly_table.dtype),
            pltpu.VMEM(daily_table.shape, daily_table.dtype),
            pltpu.SemaphoreType.DMA((_K,)),
            pltpu.SemaphoreType.DMA((_K,)),
            pltpu.SemaphoreType.DMA((S // ch,)),
            pltpu.SemaphoreType.DMA((2,)),
        ],
    )(x, pe, hourly_table, daily_table)


# speedup vs baseline: 1.8686x; 1.0015x over previous
import jax
from jax.experimental import pallas as pl
from jax.experimental.pallas import tpu as pltpu

_CH = 512
_K = 10


def _make_body(B, S, D, CH, K):
    NC = S // CH
    T = B * NC
    W = D // 4

    def body(x_hbm, pe_hbm, h_hbm, d_hbm, o_hbm,
             x_buf, o_buf, pe_buf, h_buf, d_buf,
             x_sem, o_sem, pe_sem, tbl_sem):
        def x_copy(t, slot):
            i, b = t // B, t % B
            return pltpu.make_async_copy(
                x_hbm.at[b, pl.ds(i * CH, CH), :], x_buf.at[slot], x_sem.at[slot])

        def o_copy(t, slot):
            i, b = t // B, t % B
            return pltpu.make_async_copy(
                o_buf.at[slot], o_hbm.at[b, pl.ds(i * CH, CH), :], o_sem.at[slot])

        def pe_copy(j):
            return pltpu.make_async_copy(
                pe_hbm.at[0, pl.ds(j * CH, CH), :],
                pe_buf.at[pl.ds(j * CH, CH), :], pe_sem.at[j])

        h_cp = pltpu.make_async_copy(h_hbm, h_buf, tbl_sem.at[0])
        d_cp = pltpu.make_async_copy(d_hbm, d_buf, tbl_sem.at[1])
        h_cp.start()
        d_cp.start()
        for j in range(NC):
            pe_copy(j).start()
        for t in range(min(K, T)):
            x_copy(t, t % K).start()
        h_cp.wait()
        d_cp.wait()
        bias = h_buf[0, :] + d_buf[0, :]

        for t in range(T):
            i, b = t // B, t % B
            slot = t % K
            x_copy(t, slot).wait()
            if t >= K:
                o_copy(t - K, slot).wait()
            if b == 0:
                pe_copy(i).wait()
            for k in range(4):
                sl = slice(k * W, (k + 1) * W)
                o_buf[slot, :, sl] = x_buf[slot, :, sl] + (
                    pe_buf[pl.ds(i * CH, CH), sl] + bias[None, :])
            o_copy(t, slot).start()
            if t + K < T:
                x_copy(t + K, slot).start()
        for t in range(max(T - K, 0), T):
            o_copy(t, t % K).wait()

    return body


def kernel(x, timestamps, pe, hourly_table, daily_table):
    B, S, D = x.shape
    ch = _CH if S % _CH == 0 else S
    body = _make_body(B, S, D, ch, _K)
    hbm = pl.BlockSpec(memory_space=pltpu.MemorySpace.HBM)
    return pl.pallas_call(
        body,
        in_specs=[hbm, hbm, hbm, hbm],
        out_specs=hbm,
        out_shape=jax.ShapeDtypeStruct((B, S, D), x.dtype),
        scratch_shapes=[
            pltpu.VMEM((_K, ch, D), x.dtype),
            pltpu.VMEM((_K, ch, D), x.dtype),
            pltpu.VMEM((S, D), pe.dtype),
            pltpu.VMEM(hourly_table.shape, hourly_table.dtype),
            pltpu.VMEM(daily_table.shape, daily_table.dtype),
            pltpu.SemaphoreType.DMA((_K,)),
            pltpu.SemaphoreType.DMA((_K,)),
            pltpu.SemaphoreType.DMA((S // ch,)),
            pltpu.SemaphoreType.DMA((2,)),
        ],
    )(x, pe, hourly_table, daily_table)
